# speedup vs baseline: 4.1910x; 4.1910x over previous
import jax
import jax.numpy as jnp
from jax.experimental import pallas as pl

CROP_H = 448
CROP_W = 448
ROW_BLK = 32


def _crop_body(x_ref, o_ref):
    o_ref[...] = x_ref[:, :, 32 : 32 + CROP_W]


def kernel(x):
    B, T, C, H, W = x.shape
    N = B * T * C
    top = (H - CROP_H) // 2
    xf = x.reshape(N, H, W)
    out = pl.pallas_call(
        _crop_body,
        grid=(N, CROP_H // ROW_BLK),
        in_specs=[
            pl.BlockSpec((1, ROW_BLK, W), lambda i, j: (i, j + top // ROW_BLK, 0))
        ],
        out_specs=pl.BlockSpec((1, ROW_BLK, CROP_W), lambda i, j: (i, j, 0)),
        out_shape=jax.ShapeDtypeStruct((N, CROP_H, CROP_W), x.dtype),
    )(xf)
    return out.reshape(B, T * C, CROP_H, CROP_W)


# speedup vs baseline: 22.3960x; 5.3438x over previous
import jax
import jax.numpy as jnp
from jax.experimental import pallas as pl

CROP_H = 448
CROP_W = 448
ROW_BLK = 32


def _crop_body(x_ref, o_ref):
    o_ref[...] = x_ref[:, :, 32 : 32 + CROP_W]


IMG_BLK = 8


def kernel(x):
    B, T, C, H, W = x.shape
    N = B * T * C
    top = (H - CROP_H) // 2
    xf = x.reshape(N, H, W)
    out = pl.pallas_call(
        _crop_body,
        grid=(N // IMG_BLK, CROP_H // ROW_BLK),
        in_specs=[
            pl.BlockSpec(
                (IMG_BLK, ROW_BLK, W), lambda i, j: (i, j + top // ROW_BLK, 0)
            )
        ],
        out_specs=pl.BlockSpec((IMG_BLK, ROW_BLK, CROP_W), lambda i, j: (i, j, 0)),
        out_shape=jax.ShapeDtypeStruct((N, CROP_H, CROP_W), x.dtype),
    )(xf)
    return out.reshape(B, T * C, CROP_H, CROP_W)


# speedup vs baseline: 50.0709x; 2.2357x over previous
import jax
import jax.numpy as jnp
from jax.experimental import pallas as pl

CROP_H = 448
CROP_W = 448
ROW_BLK = 32


def _crop_body(x_ref, o_ref):
    o_ref[...] = x_ref[:, :, 32 : 32 + CROP_W]


IMG_BLK = 48


def kernel(x):
    B, T, C, H, W = x.shape
    N = B * T * C
    top = (H - CROP_H) // 2
    xf = x.reshape(N, H, W)
    out = pl.pallas_call(
        _crop_body,
        grid=(N // IMG_BLK, CROP_H // ROW_BLK),
        in_specs=[
            pl.BlockSpec(
                (IMG_BLK, ROW_BLK, W), lambda i, j: (i, j + top // ROW_BLK, 0)
            )
        ],
        out_specs=pl.BlockSpec((IMG_BLK, ROW_BLK, CROP_W), lambda i, j: (i, j, 0)),
        out_shape=jax.ShapeDtypeStruct((N, CROP_H, CROP_W), x.dtype),
    )(xf)
    return out.reshape(B, T * C, CROP_H, CROP_W)


# speedup vs baseline: 53.2665x; 1.0638x over previous
import jax
import jax.numpy as jnp
from jax.experimental import pallas as pl

CROP_H = 448
CROP_W = 448
ROW_BLK = 32


def _crop_body(x_ref, o_ref):
    o_ref[...] = x_ref[:, :, 32 : 32 + CROP_W]


IMG_BLK = 96


def kernel(x):
    B, T, C, H, W = x.shape
    N = B * T * C
    top = (H - CROP_H) // 2
    xf = x.reshape(N, H, W)
    out = pl.pallas_call(
        _crop_body,
        grid=(N // IMG_BLK, CROP_H // ROW_BLK),
        in_specs=[
            pl.BlockSpec(
                (IMG_BLK, ROW_BLK, W), lambda i, j: (i, j + top // ROW_BLK, 0)
            )
        ],
        out_specs=pl.BlockSpec((IMG_BLK, ROW_BLK, CROP_W), lambda i, j: (i, j, 0)),
        out_shape=jax.ShapeDtypeStruct((N, CROP_H, CROP_W), x.dtype),
    )(xf)
    return out.reshape(B, T * C, CROP_H, CROP_W)


# speedup vs baseline: 53.6231x; 1.0067x over previous
import jax
import jax.numpy as jnp
from jax.experimental import pallas as pl

CROP_H = 448
CROP_W = 448
ROW_BLK = 32


def _crop_body(x_ref, o_ref):
    o_ref[...] = x_ref[:, :, 32 : 32 + CROP_W]


IMG_BLK = 192


def kernel(x):
    B, T, C, H, W = x.shape
    N = B * T * C
    top = (H - CROP_H) // 2
    xf = x.reshape(N, H, W)
    out = pl.pallas_call(
        _crop_body,
        grid=(N // IMG_BLK, CROP_H // ROW_BLK),
        in_specs=[
            pl.BlockSpec(
                (IMG_BLK, ROW_BLK, W), lambda i, j: (i, j + top // ROW_BLK, 0)
            )
        ],
        out_specs=pl.BlockSpec((IMG_BLK, ROW_BLK, CROP_W), lambda i, j: (i, j, 0)),
        out_shape=jax.ShapeDtypeStruct((N, CROP_H, CROP_W), x.dtype),
    )(xf)
    return out.reshape(B, T * C, CROP_H, CROP_W)
